# baseline (speedup 1.0000x reference)
import jax
import jax.numpy as jnp
from jax.experimental import pallas as pl

_BBLK = 8
_K = 64


def _fused_body(ne_ref, hs_ref, w_ref, b_ref,
                emb_ref, pol_ref, scr_ref, ent_ref, idx_ref):
    f32 = jnp.float32
    hs = hs_ref[...]
    W = w_ref[...]
    bb = b_ref[...]

    scorer = jnp.tanh(
        jax.lax.dot_general(hs, W, (((1,), (1,)), ((), ())),
                            preferred_element_type=f32) + bb)
    scr_ref[...] = scorer
    norm = jnp.sqrt(jnp.sum(scorer * scorer, axis=1, keepdims=True))

    rows = []
    for i in range(_BBLK):
        s_i = jax.lax.dot_general(scorer[i:i + 1], ne_ref[i],
                                  (((1,), (1,)), ((), ())),
                                  preferred_element_type=f32)
        rows.append(s_i)
    scores = jnp.concatenate(rows, axis=0) / norm

    G = scores.shape[1]
    iota_g = jax.lax.broadcasted_iota(jnp.int32, scores.shape, 1)
    s_work = scores
    vals_l, idx_l = [], []
    for _ in range(_K):
        m = jnp.max(s_work, axis=1, keepdims=True)
        cand = jnp.where(s_work == m, iota_g, G)
        ik = jnp.min(cand, axis=1, keepdims=True)
        vals_l.append(m)
        idx_l.append(ik)
        s_work = jnp.where(iota_g == ik, -jnp.inf, s_work)
    vals = jnp.concatenate(vals_l, axis=1)
    idxs = jnp.concatenate(idx_l, axis=1)

    m0 = vals[:, 0:1]
    e = jnp.exp(scores - m0)
    z = jnp.sum(e, axis=1, keepdims=True)
    logz = m0 + jnp.log(z)
    ps = jnp.sum(e * scores, axis=1, keepdims=True) / z
    ent_ref[...] = logz - ps
    pol_ref[...] = jnp.mean(vals, axis=1, keepdims=True) - logz

    reps = idx_ref.shape[1] // _K
    idx_ref[...] = jnp.concatenate([idxs] * reps, axis=1)

    tanh_vals = jnp.tanh(vals)
    iota_s = jax.lax.broadcasted_iota(jnp.int32, (G, _K), 0)
    for i in range(_BBLK):
        oh = (iota_s == idxs[i:i + 1, :]).astype(f32)
        g_t = jax.lax.dot_general(ne_ref[i], oh,
                                  (((0,), (0,)), ((), ())),
                                  preferred_element_type=f32)
        g_t = g_t * tanh_vals[i:i + 1, :]
        emb_ref[i] = jnp.concatenate([g_t] * reps, axis=1)


def kernel(node_embs, mask, h_selector, W, b):
    del mask
    B, G, D = node_embs.shape
    RNN = h_selector.shape[1]
    b2 = b.reshape(1, D)
    nblk = B // _BBLK

    out_shape = (
        jax.ShapeDtypeStruct((B, D, D), jnp.float32),
        jax.ShapeDtypeStruct((B, 1), jnp.float32),
        jax.ShapeDtypeStruct((B, D), jnp.float32),
        jax.ShapeDtypeStruct((B, 1), jnp.float32),
        jax.ShapeDtypeStruct((B, D), jnp.int32),
    )
    emb, pol, scr, ent, idx = pl.pallas_call(
        _fused_body,
        grid=(nblk,),
        in_specs=[
            pl.BlockSpec((_BBLK, G, D), lambda i: (i, 0, 0)),
            pl.BlockSpec((_BBLK, RNN), lambda i: (i, 0)),
            pl.BlockSpec((D, RNN), lambda i: (0, 0)),
            pl.BlockSpec((1, D), lambda i: (0, 0)),
        ],
        out_specs=[
            pl.BlockSpec((_BBLK, D, D), lambda i: (i, 0, 0)),
            pl.BlockSpec((_BBLK, 1), lambda i: (i, 0)),
            pl.BlockSpec((_BBLK, D), lambda i: (i, 0)),
            pl.BlockSpec((_BBLK, 1), lambda i: (i, 0)),
            pl.BlockSpec((_BBLK, D), lambda i: (i, 0)),
        ],
        out_shape=out_shape,
    )(node_embs, h_selector, W, b2)
    return emb, pol[:, 0], scr, ent[:, 0], idx
